# Initial kernel scaffold; baseline (speedup 1.0000x reference)
import jax
import jax.numpy as jnp
from jax.experimental import pallas as pl


def kernel(x, adj, enc_W1, enc_b1, enc_W2s, enc_b2s, enc_W2ns, enc_b2ns, ds_W1, ds_b1, ds_W2, ds_b2, dns_W1, dns_b1, dns_W2, dns_b2, sd_W1, sd_b1):
    raise NotImplementedError("write your pallas kernel here")


# speedup vs baseline: 1.6537x; 1.6537x over previous
import jax
import jax.numpy as jnp
from jax.experimental import pallas as pl
from jax.experimental.pallas import tpu as pltpu

_F32 = jnp.float32


def _mm_kern(x_ref, w_ref, o_ref):
    o_ref[...] = jnp.dot(x_ref[...], w_ref[...], preferred_element_type=_F32)


def _small_mm(x, w, bm):
    m, k = x.shape
    n = w.shape[1]
    return pl.pallas_call(
        _mm_kern,
        grid=(m // bm,),
        in_specs=[
            pl.BlockSpec((bm, k), lambda i: (i, 0)),
            pl.BlockSpec((k, n), lambda i: (0, 0)),
        ],
        out_specs=pl.BlockSpec((bm, n), lambda i: (i, 0)),
        out_shape=jax.ShapeDtypeStruct((m, n), _F32),
    )(x, w)


def _pass_s_kern(adj_ref, s_ref, b_ref, wn_ref, s_out_ref):
    h = jnp.maximum(
        jnp.dot(adj_ref[...], s_ref[...], preferred_element_type=_F32)
        + b_ref[...], 0.0)
    s_out_ref[...] = jnp.dot(h, wn_ref[...], preferred_element_type=_F32)


def _pass_hs128_kern(adj_ref, s_ref, b_ref, wn_ref, h_out_ref, s_out_ref):
    h = jnp.maximum(
        jnp.dot(adj_ref[...], s_ref[...], preferred_element_type=_F32)
        + b_ref[...], 0.0)
    h_out_ref[...] = h
    s_out_ref[...] = jnp.dot(h[:, :128], wn_ref[...],
                             preferred_element_type=_F32)


def _pass_hs_kern(adj_ref, s_ref, b_ref, wn_ref, h_out_ref, s_out_ref):
    h = jnp.maximum(
        jnp.dot(adj_ref[...], s_ref[...], preferred_element_type=_F32)
        + b_ref[...], 0.0)
    h_out_ref[...] = h
    s_out_ref[...] = jnp.dot(h, wn_ref[...], preferred_element_type=_F32)


def _pass_h_kern(adj_ref, s_ref, b_ref, h_out_ref):
    h_out_ref[...] = jnp.maximum(
        jnp.dot(adj_ref[...], s_ref[...], preferred_element_type=_F32)
        + b_ref[...], 0.0)


def _gcn_pass(adj, s, b, wnext, bm, emit_h, wcols128=False):
    n = adj.shape[0]
    w = s.shape[1]
    ni = n // bm
    in_specs = [
        pl.BlockSpec((bm, n), lambda i: (i, 0)),
        pl.BlockSpec((n, w), lambda i: (0, 0)),
        pl.BlockSpec((1, w), lambda i: (0, 0)),
    ]
    args = [adj, s, b]
    out_shapes = []
    out_specs = []
    if emit_h:
        out_shapes.append(jax.ShapeDtypeStruct((n, w), _F32))
        out_specs.append(pl.BlockSpec((bm, w), lambda i: (i, 0)))
    if wnext is not None:
        wk, wn = wnext.shape
        in_specs.append(pl.BlockSpec((wk, wn), lambda i: (0, 0)))
        args.append(wnext)
        out_shapes.append(jax.ShapeDtypeStruct((n, wn), _F32))
        out_specs.append(pl.BlockSpec((bm, wn), lambda i: (i, 0)))

    if wnext is None:
        body = _pass_h_kern
    elif not emit_h:
        body = _pass_s_kern
    else:
        body = _pass_hs128_kern if wcols128 else _pass_hs_kern

    return pl.pallas_call(
        body,
        grid=(ni,),
        in_specs=in_specs,
        out_specs=out_specs if len(out_specs) > 1 else out_specs[0],
        out_shape=out_shapes if len(out_shapes) > 1 else out_shapes[0],
        compiler_params=pltpu.CompilerParams(
            dimension_semantics=("arbitrary",)),
    )(*args)


def _struct_kern(a_ref, b_ref, o_ref):
    o_ref[...] = jax.lax.dot_general(
        a_ref[...], b_ref[...], (((1,), (1,)), ((), ())),
        preferred_element_type=_F32)


def _struct_mm(hst, bm):
    n, h = hst.shape
    return pl.pallas_call(
        _struct_kern,
        grid=(n // bm,),
        in_specs=[
            pl.BlockSpec((bm, h), lambda i: (i, 0)),
            pl.BlockSpec((n, h), lambda i: (0, 0)),
        ],
        out_specs=pl.BlockSpec((bm, n), lambda i: (i, 0)),
        out_shape=jax.ShapeDtypeStruct((n, n), _F32),
        compiler_params=pltpu.CompilerParams(
            dimension_semantics=("arbitrary",)),
    )(hst, hst)


def kernel(x, adj, enc_W1, enc_b1, enc_W2s, enc_b2s, enc_W2ns, enc_b2ns,
           ds_W1, ds_b1, ds_W2, ds_b2, dns_W1, dns_b1, dns_W2, dns_b2,
           sd_W1, sd_b1):
    hdim = enc_W1.shape[1]
    fdim = ds_W2.shape[1]
    zz = jnp.zeros((hdim, hdim), _F32)
    zz2 = jnp.zeros((hdim, fdim), _F32)
    w2cat = jnp.concatenate([enc_W2s, enc_W2ns], axis=1)
    b2cat = jnp.concatenate([enc_b2s, enc_b2ns])[None, :]
    w3cat = jnp.concatenate([
        jnp.concatenate([ds_W1, zz, zz], axis=1),
        jnp.concatenate([zz, dns_W1, sd_W1], axis=1),
    ], axis=0)
    b3cat = jnp.concatenate([ds_b1, dns_b1, sd_b1])[None, :]
    w4cat = jnp.concatenate([
        jnp.concatenate([ds_W2, zz2], axis=1),
        jnp.concatenate([zz2, dns_W2], axis=1),
    ], axis=0)
    b4cat = jnp.concatenate([ds_b2, dns_b2])[None, :]
    b1 = enc_b1[None, :]

    bm = 400
    s1 = _small_mm(x, enc_W1, bm=2000)
    s2 = _gcn_pass(adj, s1, b1, w2cat, bm, emit_h=False)
    zcat, s3 = _gcn_pass(adj, s2, b2cat, w3cat, bm, emit_h=True)
    hcat, s4 = _gcn_pass(adj, s3, b3cat, w4cat, bm, emit_h=True,
                         wcols128=True)
    xcat = _gcn_pass(adj, s4, b4cat, None, bm, emit_h=True)

    hst = hcat[:, 128:192]
    struct = _struct_mm(hst, bm=200)

    z_s, z_ns = zcat[:, :hdim], zcat[:, hdim:]
    x_s_hat, x_ns_hat = xcat[:, :fdim], xcat[:, fdim:]
    return (struct, x_s_hat, x_ns_hat, z_s, z_ns)


# speedup vs baseline: 1.8943x; 1.1455x over previous
import jax
import jax.numpy as jnp
from jax.experimental import pallas as pl
from jax.experimental.pallas import tpu as pltpu

_F32 = jnp.float32


def _mm_kern(x_ref, w_ref, o_ref):
    o_ref[...] = jnp.dot(x_ref[...], w_ref[...], preferred_element_type=_F32)


def _small_mm(x, w, bm):
    m, k = x.shape
    n = w.shape[1]
    return pl.pallas_call(
        _mm_kern,
        grid=(m // bm,),
        in_specs=[
            pl.BlockSpec((bm, k), lambda i: (i, 0)),
            pl.BlockSpec((k, n), lambda i: (0, 0)),
        ],
        out_specs=pl.BlockSpec((bm, n), lambda i: (i, 0)),
        out_shape=jax.ShapeDtypeStruct((m, n), _F32),
    )(x, w)


_BF16 = jnp.bfloat16


def _pass1_kern(adj_ref, s_ref, b_ref, wn_ref, adjbf_ref, s_out_ref):
    a = adj_ref[...]
    adjbf_ref[...] = a.astype(_BF16)
    h = jnp.maximum(
        jnp.dot(a, s_ref[...], preferred_element_type=_F32)
        + b_ref[...], 0.0)
    s_out_ref[...] = jnp.dot(h, wn_ref[...],
                             preferred_element_type=_F32).astype(_BF16)


def _pass1(adj, s, b, wnext, bm):
    n = adj.shape[0]
    w = s.shape[1]
    wn = wnext.shape[1]
    return pl.pallas_call(
        _pass1_kern,
        grid=(n // bm,),
        in_specs=[
            pl.BlockSpec((bm, n), lambda i: (i, 0)),
            pl.BlockSpec((n, w), lambda i: (0, 0)),
            pl.BlockSpec((1, w), lambda i: (0, 0)),
            pl.BlockSpec(wnext.shape, lambda i: (0, 0)),
        ],
        out_specs=[
            pl.BlockSpec((bm, n), lambda i: (i, 0)),
            pl.BlockSpec((bm, wn), lambda i: (i, 0)),
        ],
        out_shape=[
            jax.ShapeDtypeStruct((n, n), _BF16),
            jax.ShapeDtypeStruct((n, wn), _BF16),
        ],
        compiler_params=pltpu.CompilerParams(
            dimension_semantics=("arbitrary",)),
    )(adj, s, b, wnext)


def _pass_hs128_kern(adj_ref, s_ref, b_ref, wn_ref, h_out_ref, s_out_ref):
    h = jnp.maximum(
        jnp.dot(adj_ref[...], s_ref[...], preferred_element_type=_F32)
        + b_ref[...], 0.0)
    h_out_ref[...] = h
    s_out_ref[...] = jnp.dot(h[:, :128], wn_ref[...],
                             preferred_element_type=_F32).astype(_BF16)


def _pass_hs_kern(adj_ref, s_ref, b_ref, wn_ref, h_out_ref, s_out_ref):
    h = jnp.maximum(
        jnp.dot(adj_ref[...], s_ref[...], preferred_element_type=_F32)
        + b_ref[...], 0.0)
    h_out_ref[...] = h
    s_out_ref[...] = jnp.dot(h, wn_ref[...],
                             preferred_element_type=_F32).astype(_BF16)


def _pass_h_kern(adj_ref, s_ref, b_ref, h_out_ref):
    h_out_ref[...] = jnp.maximum(
        jnp.dot(adj_ref[...], s_ref[...], preferred_element_type=_F32)
        + b_ref[...], 0.0)


def _gcn_pass(adj, s, b, wnext, bm, wcols128=False):
    n = adj.shape[0]
    w = s.shape[1]
    ni = n // bm
    in_specs = [
        pl.BlockSpec((bm, n), lambda i: (i, 0)),
        pl.BlockSpec((n, w), lambda i: (0, 0)),
        pl.BlockSpec((1, w), lambda i: (0, 0)),
    ]
    args = [adj, s, b]
    out_shapes = [jax.ShapeDtypeStruct((n, w), _F32)]
    out_specs = [pl.BlockSpec((bm, w), lambda i: (i, 0))]
    if wnext is not None:
        wk, wn = wnext.shape
        in_specs.append(pl.BlockSpec((wk, wn), lambda i: (0, 0)))
        args.append(wnext)
        out_shapes.append(jax.ShapeDtypeStruct((n, wn), _BF16))
        out_specs.append(pl.BlockSpec((bm, wn), lambda i: (i, 0)))

    if wnext is None:
        body = _pass_h_kern
    else:
        body = _pass_hs128_kern if wcols128 else _pass_hs_kern

    return pl.pallas_call(
        body,
        grid=(ni,),
        in_specs=in_specs,
        out_specs=out_specs if len(out_specs) > 1 else out_specs[0],
        out_shape=out_shapes if len(out_shapes) > 1 else out_shapes[0],
        compiler_params=pltpu.CompilerParams(
            dimension_semantics=("arbitrary",)),
    )(*args)


def _struct_kern(a_ref, b_ref, o_ref):
    o_ref[...] = jax.lax.dot_general(
        a_ref[...], b_ref[...], (((1,), (1,)), ((), ())),
        preferred_element_type=_F32)


def _struct_mm(hst, bm):
    n, h = hst.shape
    return pl.pallas_call(
        _struct_kern,
        grid=(n // bm,),
        in_specs=[
            pl.BlockSpec((bm, h), lambda i: (i, 0)),
            pl.BlockSpec((n, h), lambda i: (0, 0)),
        ],
        out_specs=pl.BlockSpec((bm, n), lambda i: (i, 0)),
        out_shape=jax.ShapeDtypeStruct((n, n), _F32),
        compiler_params=pltpu.CompilerParams(
            dimension_semantics=("arbitrary",)),
    )(hst, hst)


def kernel(x, adj, enc_W1, enc_b1, enc_W2s, enc_b2s, enc_W2ns, enc_b2ns,
           ds_W1, ds_b1, ds_W2, ds_b2, dns_W1, dns_b1, dns_W2, dns_b2,
           sd_W1, sd_b1):
    hdim = enc_W1.shape[1]
    fdim = ds_W2.shape[1]
    zz = jnp.zeros((hdim, hdim), _F32)
    zz2 = jnp.zeros((hdim, fdim), _F32)
    w2cat = jnp.concatenate([enc_W2s, enc_W2ns], axis=1)
    b2cat = jnp.concatenate([enc_b2s, enc_b2ns])[None, :]
    w3cat = jnp.concatenate([
        jnp.concatenate([ds_W1, zz, zz], axis=1),
        jnp.concatenate([zz, dns_W1, sd_W1], axis=1),
    ], axis=0)
    b3cat = jnp.concatenate([ds_b1, dns_b1, sd_b1])[None, :]
    w4cat = jnp.concatenate([
        jnp.concatenate([ds_W2, zz2], axis=1),
        jnp.concatenate([zz2, dns_W2], axis=1),
    ], axis=0)
    b4cat = jnp.concatenate([ds_b2, dns_b2])[None, :]
    b1 = enc_b1[None, :]

    bm = 400
    s1 = _small_mm(x, enc_W1, bm=2000)
    adj_bf, s2 = _pass1(adj, s1, b1, w2cat, bm=200)
    zcat, s3 = _gcn_pass(adj_bf, s2, b2cat, w3cat, bm)
    hcat, s4 = _gcn_pass(adj_bf, s3, b3cat, w4cat, bm, wcols128=True)
    xcat = _gcn_pass(adj_bf, s4, b4cat, None, bm)

    hst = hcat[:, 128:192]
    struct = _struct_mm(hst, bm=200)

    z_s, z_ns = zcat[:, :hdim], zcat[:, hdim:]
    x_s_hat, x_ns_hat = xcat[:, :fdim], xcat[:, fdim:]
    return (struct, x_s_hat, x_ns_hat, z_s, z_ns)


# speedup vs baseline: 1.9804x; 1.0455x over previous
import jax
import jax.numpy as jnp
from jax.experimental import pallas as pl
from jax.experimental.pallas import tpu as pltpu

_F32 = jnp.float32


def _mm_kern(x_ref, w_ref, o_ref):
    o_ref[...] = jnp.dot(x_ref[...], w_ref[...], preferred_element_type=_F32)


def _small_mm(x, w, bm):
    m, k = x.shape
    n = w.shape[1]
    return pl.pallas_call(
        _mm_kern,
        grid=(m // bm,),
        in_specs=[
            pl.BlockSpec((bm, k), lambda i: (i, 0)),
            pl.BlockSpec((k, n), lambda i: (0, 0)),
        ],
        out_specs=pl.BlockSpec((bm, n), lambda i: (i, 0)),
        out_shape=jax.ShapeDtypeStruct((m, n), _F32),
    )(x, w)


_BF16 = jnp.bfloat16


def _pass1_kern(adj_ref, s_ref, b_ref, wn_ref, adjbf_ref, s_out_ref):
    a = adj_ref[...]
    adjbf_ref[...] = a.astype(_BF16)
    h = jnp.maximum(
        jnp.dot(a, s_ref[...], preferred_element_type=_F32)
        + b_ref[...], 0.0)
    s_out_ref[...] = jnp.dot(h, wn_ref[...],
                             preferred_element_type=_F32).astype(_BF16)


def _pass1(adj, s, b, wnext, bm):
    n = adj.shape[0]
    w = s.shape[1]
    wn = wnext.shape[1]
    return pl.pallas_call(
        _pass1_kern,
        grid=(n // bm,),
        in_specs=[
            pl.BlockSpec((bm, n), lambda i: (i, 0)),
            pl.BlockSpec((n, w), lambda i: (0, 0)),
            pl.BlockSpec((1, w), lambda i: (0, 0)),
            pl.BlockSpec(wnext.shape, lambda i: (0, 0)),
        ],
        out_specs=[
            pl.BlockSpec((bm, n), lambda i: (i, 0)),
            pl.BlockSpec((bm, wn), lambda i: (i, 0)),
        ],
        out_shape=[
            jax.ShapeDtypeStruct((n, n), _BF16),
            jax.ShapeDtypeStruct((n, wn), _BF16),
        ],
        compiler_params=pltpu.CompilerParams(
            dimension_semantics=("arbitrary",)),
    )(adj, s, b, wnext)


def _pass_hs128_kern(adj_ref, s_ref, b_ref, wn_ref, h_out_ref, s_out_ref):
    h = jnp.maximum(
        jnp.dot(adj_ref[...], s_ref[...], preferred_element_type=_F32)
        + b_ref[...], 0.0)
    h_out_ref[...] = h
    s_out_ref[...] = jnp.dot(h[:, :128], wn_ref[...],
                             preferred_element_type=_F32).astype(_BF16)


def _pass_hs_kern(adj_ref, s_ref, b_ref, wn_ref, h_out_ref, s_out_ref):
    h = jnp.maximum(
        jnp.dot(adj_ref[...], s_ref[...], preferred_element_type=_F32)
        + b_ref[...], 0.0)
    h_out_ref[...] = h
    s_out_ref[...] = jnp.dot(h, wn_ref[...],
                             preferred_element_type=_F32).astype(_BF16)


def _pass_h_kern(adj_ref, s_ref, b_ref, h_out_ref):
    h_out_ref[...] = jnp.maximum(
        jnp.dot(adj_ref[...], s_ref[...], preferred_element_type=_F32)
        + b_ref[...], 0.0)


def _gcn_pass(adj, s, b, wnext, bm, wcols128=False):
    n = adj.shape[0]
    w = s.shape[1]
    ni = n // bm
    in_specs = [
        pl.BlockSpec((bm, n), lambda i: (i, 0)),
        pl.BlockSpec((n, w), lambda i: (0, 0)),
        pl.BlockSpec((1, w), lambda i: (0, 0)),
    ]
    args = [adj, s, b]
    out_shapes = [jax.ShapeDtypeStruct((n, w), _F32)]
    out_specs = [pl.BlockSpec((bm, w), lambda i: (i, 0))]
    if wnext is not None:
        wk, wn = wnext.shape
        in_specs.append(pl.BlockSpec((wk, wn), lambda i: (0, 0)))
        args.append(wnext)
        out_shapes.append(jax.ShapeDtypeStruct((n, wn), _BF16))
        out_specs.append(pl.BlockSpec((bm, wn), lambda i: (i, 0)))

    if wnext is None:
        body = _pass_h_kern
    else:
        body = _pass_hs128_kern if wcols128 else _pass_hs_kern

    return pl.pallas_call(
        body,
        grid=(ni,),
        in_specs=in_specs,
        out_specs=out_specs if len(out_specs) > 1 else out_specs[0],
        out_shape=out_shapes if len(out_shapes) > 1 else out_shapes[0],
        compiler_params=pltpu.CompilerParams(
            dimension_semantics=("arbitrary",)),
    )(*args)


def _struct_kern(a_ref, b_ref, o_ref):
    o_ref[...] = jax.lax.dot_general(
        a_ref[...], b_ref[...], (((1,), (1,)), ((), ())),
        preferred_element_type=_F32)


def _struct_mm(hst, bm):
    n, h = hst.shape
    return pl.pallas_call(
        _struct_kern,
        grid=(n // bm,),
        in_specs=[
            pl.BlockSpec((bm, h), lambda i: (i, 0)),
            pl.BlockSpec((n, h), lambda i: (0, 0)),
        ],
        out_specs=pl.BlockSpec((bm, n), lambda i: (i, 0)),
        out_shape=jax.ShapeDtypeStruct((n, n), _F32),
        compiler_params=pltpu.CompilerParams(
            dimension_semantics=("arbitrary",)),
    )(hst, hst)


def kernel(x, adj, enc_W1, enc_b1, enc_W2s, enc_b2s, enc_W2ns, enc_b2ns,
           ds_W1, ds_b1, ds_W2, ds_b2, dns_W1, dns_b1, dns_W2, dns_b2,
           sd_W1, sd_b1):
    hdim = enc_W1.shape[1]
    fdim = ds_W2.shape[1]
    zz = jnp.zeros((hdim, hdim), _F32)
    zz2 = jnp.zeros((hdim, fdim), _F32)
    w2cat = jnp.concatenate([enc_W2s, enc_W2ns], axis=1)
    b2cat = jnp.concatenate([enc_b2s, enc_b2ns])[None, :]
    w3cat = jnp.concatenate([
        jnp.concatenate([ds_W1, zz, zz], axis=1),
        jnp.concatenate([zz, dns_W1, sd_W1], axis=1),
    ], axis=0)
    b3cat = jnp.concatenate([ds_b1, dns_b1, sd_b1])[None, :]
    w4cat = jnp.concatenate([
        jnp.concatenate([ds_W2, zz2], axis=1),
        jnp.concatenate([zz2, dns_W2], axis=1),
    ], axis=0)
    b4cat = jnp.concatenate([ds_b2, dns_b2])[None, :]
    b1 = enc_b1[None, :]

    bm = 1000
    s1 = _small_mm(x, enc_W1, bm=2000)
    adj_bf, s2 = _pass1(adj, s1, b1, w2cat, bm=400)
    zcat, s3 = _gcn_pass(adj_bf, s2, b2cat, w3cat, bm)
    hcat, s4 = _gcn_pass(adj_bf, s3, b3cat, w4cat, bm, wcols128=True)
    xcat = _gcn_pass(adj_bf, s4, b4cat, None, bm)

    hst = hcat[:, 128:192]
    struct = _struct_mm(hst, bm=400)

    z_s, z_ns = zcat[:, :hdim], zcat[:, hdim:]
    x_s_hat, x_ns_hat = xcat[:, :fdim], xcat[:, fdim:]
    return (struct, x_s_hat, x_ns_hat, z_s, z_ns)
